# baseline (speedup 1.0000x reference)
import jax
import jax.numpy as jnp
from jax import lax
from jax.experimental import pallas as pl
from jax.experimental.pallas import tpu as pltpu
from jax.experimental.pallas import tpu_sc as plsc

B = 4096
SEQ = 50
VOCAB = 256
EMB = 64
H1 = 128
H2 = 64

NC = 2
NS = 16
NW = NC * NS
SPW = B // NW
PAIR = 2
ROWS = SEQ * PAIR
NCHUNK = SPW // PAIR
FV = H1 // 16


def _fold_kernel(table_ref, w1_ref, b1_ref, out_ref):
    tw = lax.dot_general(
        table_ref[...],
        w1_ref[...],
        (((1,), (1,)), ((), ())),
        preferred_element_type=jnp.float32,
    )
    out_ref[...] = tw + b1_ref[0][None, None, :] * (1.0 / SEQ)


def _tail_kernel(h1_ref, w2_ref, b2_ref, w3_ref, b3_ref, out_ref):
    x1 = jnp.maximum(h1_ref[...], 0.0)
    x2 = jnp.maximum(
        jnp.dot(x1, w2_ref[...], preferred_element_type=jnp.float32)
        + b2_ref[...],
        0.0,
    )
    out_ref[...] = (
        jnp.dot(x2, w3_ref[...], preferred_element_type=jnp.float32) + b3_ref[...]
    )


NBUF = 8


def _sc_gather_sum(idx_hbm, tw_hbm, out_hbm, idx_v, rows_v, out_v, *sems):
    wid = lax.axis_index("s") * NC + lax.axis_index("c")
    base = wid * SPW

    pltpu.sync_copy(idx_hbm.at[wid], idx_v)

    def start(c, b):
        pltpu.make_async_copy(tw_hbm.at[idx_v.at[c]], rows_v.at[b], sems[b]).start()

    def wait(c, b):
        pltpu.make_async_copy(tw_hbm.at[idx_v.at[c]], rows_v.at[b], sems[b]).wait()

    for b in range(NBUF):
        start(b, b)

    def body(i, _):
        for b in range(NBUF):
            c = i * NBUF + b
            wait(c, b)
            for p in range(PAIR):
                RB = 10

                def rbody(j, accs, p=p):
                    r0 = p * SEQ + j * RB
                    for rr in range(RB):
                        accs = [
                            accs[f] + rows_v[b, r0 + rr, pl.ds(f * 16, 16)]
                            for f in range(FV)
                        ]
                    return accs

                zero = jnp.zeros((16,), jnp.float32)
                accs = lax.fori_loop(0, SEQ // RB, rbody, [zero] * FV)
                for f in range(FV):
                    out_v[c * PAIR + p, pl.ds(f * 16, 16)] = accs[f]

            cn = c + NBUF

            @pl.when(cn < NCHUNK)
            def _():
                start(cn, b)

        return 0

    lax.fori_loop(0, NCHUNK // NBUF, body, 0)

    pltpu.sync_copy(out_v, out_hbm.at[pl.ds(base, SPW)])


@jax.jit
def kernel(inputs, table, W1, b1, W2, b2, W3, b3):
    ids = inputs.astype(jnp.int32)

    w1r = W1.reshape(SEQ, EMB, H1)
    b1r = b1.reshape(1, H1)
    tw = pl.pallas_call(
        _fold_kernel,
        in_specs=[
            pl.BlockSpec((VOCAB, EMB), lambda: (0, 0)),
            pl.BlockSpec((SEQ, EMB, H1), lambda: (0, 0, 0)),
            pl.BlockSpec((1, H1), lambda: (0, 0)),
        ],
        out_specs=pl.BlockSpec((VOCAB, SEQ, H1), lambda: (0, 0, 0)),
        out_shape=jax.ShapeDtypeStruct((VOCAB, SEQ, H1), jnp.float32),
    )(table, w1r, b1r)
    tw = tw.reshape(VOCAB * SEQ, H1)

    offs = jnp.arange(SEQ, dtype=jnp.int32)[None, :]
    idx = (ids * SEQ + offs).reshape(NW, NCHUNK, ROWS)

    mesh = plsc.VectorSubcoreMesh(core_axis_name="c", subcore_axis_name="s")
    h1 = pl.kernel(
        _sc_gather_sum,
        out_type=jax.ShapeDtypeStruct((B, H1), jnp.float32),
        mesh=mesh,
        scratch_types=[
            pltpu.VMEM((NCHUNK, ROWS), jnp.int32),
            pltpu.VMEM((NBUF, ROWS, H1), jnp.float32),
            pltpu.VMEM((SPW, H1), jnp.float32),
        ]
        + [pltpu.SemaphoreType.DMA] * NBUF,
    )(idx, tw)

    out = pl.pallas_call(
        _tail_kernel,
        in_specs=[
            pl.BlockSpec((B, H1), lambda: (0, 0)),
            pl.BlockSpec((H1, H2), lambda: (0, 0)),
            pl.BlockSpec((1, H2), lambda: (0, 0)),
            pl.BlockSpec((H2, 1), lambda: (0, 0)),
            pl.BlockSpec((1, 1), lambda: (0, 0)),
        ],
        out_specs=pl.BlockSpec((B, 1), lambda: (0, 0)),
        out_shape=jax.ShapeDtypeStruct((B, 1), jnp.float32),
    )(h1, W2, b2.reshape(1, H2), W3, b3.reshape(1, 1))
    return out


# speedup vs baseline: 1.1127x; 1.1127x over previous
import jax
import jax.numpy as jnp
from jax import lax
from jax.experimental import pallas as pl
from jax.experimental.pallas import tpu as pltpu
from jax.experimental.pallas import tpu_sc as plsc

B = 4096
SEQ = 50
VOCAB = 256
EMB = 64
H1 = 128
H2 = 64

NC = 2
NS = 16
NW = NC * NS
SPW = B // NW
PAIR = 2
ROWS = SEQ * PAIR
NCHUNK = SPW // PAIR
FV = H1 // 16


def _fold_kernel(table_ref, w1_ref, b1_ref, out_ref):
    tw = lax.dot_general(
        table_ref[...],
        w1_ref[...],
        (((1,), (1,)), ((), ())),
        preferred_element_type=jnp.float32,
    )
    out_ref[...] = tw + b1_ref[0][None, None, :] * (1.0 / SEQ)


def _tail_kernel(h1_ref, w2_ref, b2_ref, w3_ref, b3_ref, out_ref):
    x1 = jnp.maximum(h1_ref[...], 0.0)
    x2 = jnp.maximum(
        jnp.dot(x1, w2_ref[...], preferred_element_type=jnp.float32)
        + b2_ref[...],
        0.0,
    )
    out_ref[...] = (
        jnp.dot(x2, w3_ref[...], preferred_element_type=jnp.float32) + b3_ref[...]
    )


NBUF = 4


def _sc_gather_sum(idx_hbm, tw_hbm, out_hbm, idx_v, rows_v, out_v, *sems):
    wid = lax.axis_index("s") * NC + lax.axis_index("c")
    base = wid * SPW

    pltpu.sync_copy(idx_hbm.at[wid], idx_v)

    def start(c, b):
        pltpu.make_async_copy(tw_hbm.at[idx_v.at[c]], rows_v.at[b], sems[b]).start()

    def wait(c, b):
        pltpu.make_async_copy(tw_hbm.at[idx_v.at[c]], rows_v.at[b], sems[b]).wait()

    for b in range(NBUF):
        start(b, b)

    def body(i, _):
        for b in range(NBUF):
            c = i * NBUF + b
            wait(c, b)
            for p in range(PAIR):
                RB = 10

                def rbody(j, accs, p=p):
                    r0 = p * SEQ + j * RB
                    for rr in range(RB):
                        accs = [
                            accs[f] + rows_v[b, r0 + rr, pl.ds(f * 16, 16)]
                            for f in range(FV)
                        ]
                    return accs

                zero = jnp.zeros((16,), jnp.float32)
                accs = lax.fori_loop(0, SEQ // RB, rbody, [zero] * FV)
                for f in range(FV):
                    out_v[c * PAIR + p, pl.ds(f * 16, 16)] = accs[f]

            cn = c + NBUF

            @pl.when(cn < NCHUNK)
            def _():
                start(cn, b)

        return 0

    lax.fori_loop(0, NCHUNK // NBUF, body, 0)

    pltpu.sync_copy(out_v, out_hbm.at[pl.ds(base, SPW)])


@jax.jit
def kernel(inputs, table, W1, b1, W2, b2, W3, b3):
    ids = inputs.astype(jnp.int32)

    w1r = W1.reshape(SEQ, EMB, H1)
    b1r = b1.reshape(1, H1)
    tw = pl.pallas_call(
        _fold_kernel,
        in_specs=[
            pl.BlockSpec((VOCAB, EMB), lambda: (0, 0)),
            pl.BlockSpec((SEQ, EMB, H1), lambda: (0, 0, 0)),
            pl.BlockSpec((1, H1), lambda: (0, 0)),
        ],
        out_specs=pl.BlockSpec((VOCAB, SEQ, H1), lambda: (0, 0, 0)),
        out_shape=jax.ShapeDtypeStruct((VOCAB, SEQ, H1), jnp.float32),
    )(table, w1r, b1r)
    tw = tw.reshape(VOCAB * SEQ, H1)

    offs = jnp.arange(SEQ, dtype=jnp.int32)[None, :]
    idx = (ids * SEQ + offs).reshape(NW, NCHUNK, ROWS)

    mesh = plsc.VectorSubcoreMesh(core_axis_name="c", subcore_axis_name="s")
    h1 = pl.kernel(
        _sc_gather_sum,
        out_type=jax.ShapeDtypeStruct((B, H1), jnp.float32),
        mesh=mesh,
        scratch_types=[
            pltpu.VMEM((NCHUNK, ROWS), jnp.int32),
            pltpu.VMEM((NBUF, ROWS, H1), jnp.float32),
            pltpu.VMEM((SPW, H1), jnp.float32),
        ]
        + [pltpu.SemaphoreType.DMA] * NBUF,
    )(idx, tw)

    out = pl.pallas_call(
        _tail_kernel,
        in_specs=[
            pl.BlockSpec((B, H1), lambda: (0, 0)),
            pl.BlockSpec((H1, H2), lambda: (0, 0)),
            pl.BlockSpec((1, H2), lambda: (0, 0)),
            pl.BlockSpec((H2, 1), lambda: (0, 0)),
            pl.BlockSpec((1, 1), lambda: (0, 0)),
        ],
        out_specs=pl.BlockSpec((B, 1), lambda: (0, 0)),
        out_shape=jax.ShapeDtypeStruct((B, 1), jnp.float32),
    )(h1, W2, b2.reshape(1, H2), W3, b3.reshape(1, 1))
    return out


# speedup vs baseline: 1.1270x; 1.0129x over previous
import jax
import jax.numpy as jnp
from jax import lax
from jax.experimental import pallas as pl
from jax.experimental.pallas import tpu as pltpu
from jax.experimental.pallas import tpu_sc as plsc

B = 4096
SEQ = 50
VOCAB = 256
EMB = 64
H1 = 128
H2 = 64

NC = 2
NS = 16
NW = NC * NS
SPW = B // NW
PAIR = 1
ROWS = SEQ * PAIR
NCHUNK = SPW // PAIR
FV = H1 // 16


def _fold_kernel(table_ref, w1_ref, b1_ref, out_ref):
    tw = lax.dot_general(
        table_ref[...],
        w1_ref[...],
        (((1,), (1,)), ((), ())),
        preferred_element_type=jnp.float32,
    )
    out_ref[...] = tw + b1_ref[0][None, None, :] * (1.0 / SEQ)


def _tail_kernel(h1_ref, w2_ref, b2_ref, w3_ref, b3_ref, out_ref):
    x1 = jnp.maximum(h1_ref[...], 0.0)
    x2 = jnp.maximum(
        jnp.dot(x1, w2_ref[...], preferred_element_type=jnp.float32)
        + b2_ref[...],
        0.0,
    )
    out_ref[...] = (
        jnp.dot(x2, w3_ref[...], preferred_element_type=jnp.float32) + b3_ref[...]
    )


NBUF = 8


def _sc_gather_sum(idx_hbm, tw_hbm, out_hbm, idx_v, rows_v, out_v, *sems):
    wid = lax.axis_index("s") * NC + lax.axis_index("c")
    base = wid * SPW

    pltpu.sync_copy(idx_hbm.at[wid], idx_v)

    def start(c, b):
        pltpu.make_async_copy(tw_hbm.at[idx_v.at[c]], rows_v.at[b], sems[b]).start()

    def wait(c, b):
        pltpu.make_async_copy(tw_hbm.at[idx_v.at[c]], rows_v.at[b], sems[b]).wait()

    for b in range(NBUF):
        start(b, b)

    def body(i, _):
        for b in range(NBUF):
            c = i * NBUF + b
            wait(c, b)
            for p in range(PAIR):
                RB = 10

                def rbody(j, accs, p=p):
                    r0 = p * SEQ + j * RB
                    for rr in range(RB):
                        accs = [
                            accs[f] + rows_v[b, r0 + rr, pl.ds(f * 16, 16)]
                            for f in range(FV)
                        ]
                    return accs

                zero = jnp.zeros((16,), jnp.float32)
                accs = lax.fori_loop(0, SEQ // RB, rbody, [zero] * FV)
                for f in range(FV):
                    out_v[c * PAIR + p, pl.ds(f * 16, 16)] = accs[f]

            cn = c + NBUF

            @pl.when(cn < NCHUNK)
            def _():
                start(cn, b)

        return 0

    lax.fori_loop(0, NCHUNK // NBUF, body, 0)

    pltpu.sync_copy(out_v, out_hbm.at[pl.ds(base, SPW)])


@jax.jit
def kernel(inputs, table, W1, b1, W2, b2, W3, b3):
    ids = inputs.astype(jnp.int32)

    w1r = W1.reshape(SEQ, EMB, H1)
    b1r = b1.reshape(1, H1)
    tw = pl.pallas_call(
        _fold_kernel,
        in_specs=[
            pl.BlockSpec((VOCAB, EMB), lambda: (0, 0)),
            pl.BlockSpec((SEQ, EMB, H1), lambda: (0, 0, 0)),
            pl.BlockSpec((1, H1), lambda: (0, 0)),
        ],
        out_specs=pl.BlockSpec((VOCAB, SEQ, H1), lambda: (0, 0, 0)),
        out_shape=jax.ShapeDtypeStruct((VOCAB, SEQ, H1), jnp.float32),
    )(table, w1r, b1r)
    tw = tw.reshape(VOCAB * SEQ, H1)

    offs = jnp.arange(SEQ, dtype=jnp.int32)[None, :]
    idx = (ids * SEQ + offs).reshape(NW, NCHUNK, ROWS)

    mesh = plsc.VectorSubcoreMesh(core_axis_name="c", subcore_axis_name="s")
    h1 = pl.kernel(
        _sc_gather_sum,
        out_type=jax.ShapeDtypeStruct((B, H1), jnp.float32),
        mesh=mesh,
        scratch_types=[
            pltpu.VMEM((NCHUNK, ROWS), jnp.int32),
            pltpu.VMEM((NBUF, ROWS, H1), jnp.float32),
            pltpu.VMEM((SPW, H1), jnp.float32),
        ]
        + [pltpu.SemaphoreType.DMA] * NBUF,
    )(idx, tw)

    out = pl.pallas_call(
        _tail_kernel,
        in_specs=[
            pl.BlockSpec((B, H1), lambda: (0, 0)),
            pl.BlockSpec((H1, H2), lambda: (0, 0)),
            pl.BlockSpec((1, H2), lambda: (0, 0)),
            pl.BlockSpec((H2, 1), lambda: (0, 0)),
            pl.BlockSpec((1, 1), lambda: (0, 0)),
        ],
        out_specs=pl.BlockSpec((B, 1), lambda: (0, 0)),
        out_shape=jax.ShapeDtypeStruct((B, 1), jnp.float32),
    )(h1, W2, b2.reshape(1, H2), W3, b3.reshape(1, 1))
    return out
